# Initial kernel scaffold; baseline (speedup 1.0000x reference)
import functools

import jax
import jax.numpy as jnp
from jax.experimental import pallas as pl
from jax.experimental.pallas import tpu as pltpu

E = 8
TOP_K = 2
H = 1024
I = 1024
N = 2048
TM = 512


def _moe_body(x_ref, gw_ref, alpha_ref, f1w_ref, f1b_ref, f2w_ref, f2b_ref,
              out_ref, acc_ref):
    e = pl.program_id(1)
    x = x_ref[...]

    logits = jnp.dot(x, gw_ref[...], preferred_element_type=jnp.float32)
    probs = jax.nn.softmax(logits, axis=-1)
    m1 = jnp.max(probs, axis=-1, keepdims=True)
    masked = jnp.where(probs >= m1, -1.0, probs)
    m2 = jnp.max(masked, axis=-1, keepdims=True)
    sel = probs >= m2
    lane = jax.lax.broadcasted_iota(jnp.int32, probs.shape, 1)
    w_e = jnp.sum(jnp.where(sel & (lane == e), probs, 0.0), axis=-1)
    coef = (w_e * alpha_ref[e])[:, None]

    xb = x.astype(jnp.bfloat16)
    h1 = jnp.dot(xb, f1w_ref[0], preferred_element_type=jnp.float32)
    h1 = h1 + f1b_ref[0, 0, :][None, :]
    g = jax.nn.gelu(h1, approximate=False)
    y = jnp.dot(g.astype(jnp.bfloat16), f2w_ref[0],
                preferred_element_type=jnp.float32)
    y = y + f2b_ref[0, 0, :][None, :]
    contrib = y * coef

    @pl.when(e == 0)
    def _():
        acc_ref[...] = contrib

    @pl.when(e > 0)
    def _():
        acc_ref[...] = acc_ref[...] + contrib

    @pl.when(e == E - 1)
    def _():
        out_ref[...] = acc_ref[...]


@jax.jit
def _moe(flat, gate_w, alpha, f1w, f1b, f2w, f2b):
    nt = N // TM
    grid = (nt, E)
    return pl.pallas_call(
        _moe_body,
        grid=grid,
        in_specs=[
            pl.BlockSpec((TM, H), lambda t, e: (t, 0)),
            pl.BlockSpec((H, E), lambda t, e: (0, 0)),
            pl.BlockSpec(memory_space=pltpu.SMEM),
            pl.BlockSpec((1, H, I), lambda t, e: (e, 0, 0)),
            pl.BlockSpec((1, 1, I), lambda t, e: (e, 0, 0)),
            pl.BlockSpec((1, I, H), lambda t, e: (e, 0, 0)),
            pl.BlockSpec((1, 1, H), lambda t, e: (e, 0, 0)),
        ],
        out_specs=pl.BlockSpec((TM, H), lambda t, e: (t, 0)),
        out_shape=jax.ShapeDtypeStruct((N, H), jnp.float32),
        scratch_shapes=[pltpu.VMEM((TM, H), jnp.float32)],
    )(flat, gate_w, alpha, f1w, f1b, f2w, f2b)


def kernel(hidden_states, gate_w, fc1_w, fc1_b, fc2_w, fc2_b, alpha):
    b, s, h = hidden_states.shape
    flat = hidden_states.reshape(-1, h)
    f1w = fc1_w.astype(jnp.bfloat16)
    f2w = fc2_w.astype(jnp.bfloat16)
    f1b = fc1_b.reshape(E, 1, I)
    f2b = fc2_b.reshape(E, 1, H)
    out = _moe(flat, gate_w, alpha, f1w, f1b, f2w, f2b)
    return out.reshape(b, s, h)


# speedup vs baseline: 2.1415x; 2.1415x over previous
import functools

import jax
import jax.numpy as jnp
from jax.experimental import pallas as pl
from jax.experimental.pallas import tpu as pltpu

E = 8
TOP_K = 2
H = 1024
I = 1024
N = 2048
TM = 512


def _moe_body(x_ref, gw_ref, alpha_ref, f1w_ref, f1b_ref, f2w_ref, f2b_ref,
              out_ref, acc_ref):
    e = pl.program_id(1)
    x = x_ref[...]

    logits = jnp.dot(x, gw_ref[...], preferred_element_type=jnp.float32)
    probs = jax.nn.softmax(logits, axis=-1)
    m1 = jnp.max(probs, axis=-1, keepdims=True)
    masked = jnp.where(probs >= m1, -1.0, probs)
    m2 = jnp.max(masked, axis=-1, keepdims=True)
    sel = probs >= m2
    lane = jax.lax.broadcasted_iota(jnp.int32, probs.shape, 1)
    w_e = jnp.sum(jnp.where(sel & (lane == e), probs, 0.0), axis=-1)
    coef = (w_e * alpha_ref[e])[:, None]

    xb = x.astype(jnp.bfloat16)
    h1 = jnp.dot(xb, f1w_ref[0], preferred_element_type=jnp.float32)
    h1 = h1 + f1b_ref[0, 0, :][None, :]
    g = 0.5 * h1 * (1.0 + jax.lax.erf(h1 * 0.7071067811865476))
    y = jnp.dot(g.astype(jnp.bfloat16), f2w_ref[0],
                preferred_element_type=jnp.float32)
    y = y + f2b_ref[0, 0, :][None, :]
    contrib = y * coef

    @pl.when(e == 0)
    def _():
        acc_ref[...] = contrib

    @pl.when(e > 0)
    def _():
        acc_ref[...] = acc_ref[...] + contrib

    @pl.when(e == E - 1)
    def _():
        out_ref[...] = acc_ref[...]


@jax.jit
def _moe(flat, gate_w, alpha, f1w, f1b, f2w, f2b):
    nt = N // TM
    grid = (nt, E)
    return pl.pallas_call(
        _moe_body,
        grid=grid,
        in_specs=[
            pl.BlockSpec((TM, H), lambda t, e: (t, 0)),
            pl.BlockSpec((H, E), lambda t, e: (0, 0)),
            pl.BlockSpec(memory_space=pltpu.SMEM),
            pl.BlockSpec((1, H, I), lambda t, e: (e, 0, 0)),
            pl.BlockSpec((1, 1, I), lambda t, e: (e, 0, 0)),
            pl.BlockSpec((1, I, H), lambda t, e: (e, 0, 0)),
            pl.BlockSpec((1, 1, H), lambda t, e: (e, 0, 0)),
        ],
        out_specs=pl.BlockSpec((TM, H), lambda t, e: (t, 0)),
        out_shape=jax.ShapeDtypeStruct((N, H), jnp.float32),
        scratch_shapes=[pltpu.VMEM((TM, H), jnp.float32)],
    )(flat, gate_w, alpha, f1w, f1b, f2w, f2b)


def kernel(hidden_states, gate_w, fc1_w, fc1_b, fc2_w, fc2_b, alpha):
    b, s, h = hidden_states.shape
    flat = hidden_states.reshape(-1, h)
    f1w = fc1_w.astype(jnp.bfloat16)
    f2w = fc2_w.astype(jnp.bfloat16)
    f1b = fc1_b.reshape(E, 1, I)
    f2b = fc2_b.reshape(E, 1, H)
    out = _moe(flat, gate_w, alpha, f1w, f1b, f2w, f2b)
    return out.reshape(b, s, h)
